# baseline (speedup 1.0000x reference)
import functools

import jax
import jax.numpy as jnp
from jax import lax
from jax.experimental import pallas as pl
from jax.experimental.pallas import tpu as pltpu


def _round_up(x, m):
    return (x + m - 1) // m * m


def _edge_gather_kernel(idx_ref, s_ref, t_ref, out_ref, idx_smem, prod_buf,
                        sem, *, te, p, g2):
    c = pl.program_id(0)
    j = pl.program_id(1)
    row = c * g2 + j
    slot = lax.rem(j, 2)
    nslot = lax.rem(j + 1, 2)

    @pl.when(j == 0)
    def _():
        pltpu.make_async_copy(idx_ref.at[row], idx_smem.at[slot],
                              sem.at[slot]).start()

    @pl.when(j + 1 < g2)
    def _():
        pltpu.make_async_copy(idx_ref.at[row + 1], idx_smem.at[nslot],
                              sem.at[nslot]).start()

    pltpu.make_async_copy(idx_ref.at[row], idx_smem.at[slot],
                          sem.at[slot]).wait()

    def gather_loop(k):
        for mi in range(te):
            a = pl.multiple_of(idx_smem[k, 0, mi], p)
            b = pl.multiple_of(idx_smem[k, 0, te + mi], p)
            slab = s_ref[pl.ds(a, p), :] * t_ref[pl.ds(b, p), :]
            prod_buf[mi:mi + 1, :] = jnp.sum(slab, axis=0, keepdims=True)

    @pl.when(slot == 0)
    def _():
        gather_loop(0)

    @pl.when(slot == 1)
    def _():
        gather_loop(1)

    ones_row = jnp.ones((1, 128), dtype=jnp.float32)
    val = lax.dot_general(ones_row, prod_buf[...], (((1,), (1,)), ((), ())),
                          preferred_element_type=jnp.float32)
    out_ref[...] = jax.nn.sigmoid(val)[0]


def kernel(s, t, edge_index, edge_tile=8192):
    n, d = s.shape
    e = edge_index.shape[1]
    s = s.astype(jnp.float32)
    t = t.astype(jnp.float32)
    assert d % 128 == 0, "embedding dim must be lane-aligned"
    p = d // 128

    te = edge_tile
    e_pad = _round_up(max(e, 1), 2 * te)
    g = e_pad // te
    g2 = g // 2

    src = jnp.zeros((e_pad,), jnp.int32).at[:e].set(
        edge_index[0].astype(jnp.int32) * p)
    dst = jnp.zeros((e_pad,), jnp.int32).at[:e].set(
        edge_index[1].astype(jnp.int32) * p)
    idx = jnp.concatenate([src.reshape(g, 1, te), dst.reshape(g, 1, te)],
                          axis=2)

    s4 = s.reshape(n * p, 128)
    t4 = t.reshape(n * p, 128)

    body = functools.partial(_edge_gather_kernel, te=te, p=p, g2=g2)

    out = pl.pallas_call(
        body,
        out_shape=jax.ShapeDtypeStruct((e_pad,), jnp.float32),
        grid_spec=pltpu.PrefetchScalarGridSpec(
            num_scalar_prefetch=0,
            grid=(2, g2),
            in_specs=[
                pl.BlockSpec(memory_space=pltpu.VMEM),
                pl.BlockSpec(memory_space=pltpu.VMEM),
                pl.BlockSpec(memory_space=pltpu.VMEM),
            ],
            out_specs=pl.BlockSpec((te,), lambda c, j: (c * g2 + j,)),
            scratch_shapes=[
                pltpu.SMEM((2, 1, 2 * te), jnp.int32),
                pltpu.VMEM((te, 128), jnp.float32),
                pltpu.SemaphoreType.DMA((2,)),
            ],
        ),
        compiler_params=pltpu.CompilerParams(
            dimension_semantics=("parallel", "arbitrary")),
        cost_estimate=pl.CostEstimate(
            flops=2 * e_pad * d,
            transcendentals=e_pad,
            bytes_accessed=4 * (2 * n * d + 2 * e_pad * d + 3 * e_pad)),
    )(idx, s4, t4)
    return out[:e]


# speedup vs baseline: 1.2276x; 1.2276x over previous
import functools

import jax
import jax.numpy as jnp
from jax import lax
from jax.experimental import pallas as pl
from jax.experimental.pallas import tpu as pltpu


def _round_up(x, m):
    return (x + m - 1) // m * m


def _edge_gather_kernel(idx_ref, s_ref, t_ref, out_ref, idx_smem, prod_buf,
                        sem, *, te, p, g2):
    c = pl.program_id(0)
    j = pl.program_id(1)
    row = c * g2 + j
    slot = lax.rem(j, 2)
    nslot = lax.rem(j + 1, 2)

    @pl.when(j == 0)
    def _():
        pltpu.make_async_copy(idx_ref.at[row], idx_smem.at[slot],
                              sem.at[slot]).start()

    @pl.when(j + 1 < g2)
    def _():
        pltpu.make_async_copy(idx_ref.at[row + 1], idx_smem.at[nslot],
                              sem.at[nslot]).start()

    pltpu.make_async_copy(idx_ref.at[row], idx_smem.at[slot],
                          sem.at[slot]).wait()

    def gather_loop(k):
        for mi in range(te):
            a = pl.multiple_of(idx_smem[k, 0, mi], p)
            b = pl.multiple_of(idx_smem[k, 0, te + mi], p)
            slab = s_ref[pl.ds(a, p), :] * t_ref[pl.ds(b, p), :]
            prod_buf[p * mi:p * mi + p, :] = slab

    @pl.when(slot == 0)
    def _():
        gather_loop(0)

    @pl.when(slot == 1)
    def _():
        gather_loop(1)

    acc = prod_buf[0:p * te:p, :]
    for k in range(1, p):
        acc = acc + prod_buf[k:k + p * te:p, :]
    ones_row = jnp.ones((1, 128), dtype=jnp.float32)
    val = lax.dot_general(ones_row, acc, (((1,), (1,)), ((), ())),
                          preferred_element_type=jnp.float32)
    out_ref[...] = jax.nn.sigmoid(val)[0]


def kernel(s, t, edge_index, edge_tile=8192):
    n, d = s.shape
    e = edge_index.shape[1]
    s = s.astype(jnp.float32)
    t = t.astype(jnp.float32)
    assert d % 128 == 0, "embedding dim must be lane-aligned"
    p = d // 128

    te = edge_tile
    e_pad = _round_up(max(e, 1), 2 * te)
    g = e_pad // te
    g2 = g // 2

    src = jnp.zeros((e_pad,), jnp.int32).at[:e].set(
        edge_index[0].astype(jnp.int32) * p)
    dst = jnp.zeros((e_pad,), jnp.int32).at[:e].set(
        edge_index[1].astype(jnp.int32) * p)
    idx = jnp.concatenate([src.reshape(g, 1, te), dst.reshape(g, 1, te)],
                          axis=2)

    s4 = s.reshape(n * p, 128)
    t4 = t.reshape(n * p, 128)

    body = functools.partial(_edge_gather_kernel, te=te, p=p, g2=g2)

    out = pl.pallas_call(
        body,
        out_shape=jax.ShapeDtypeStruct((e_pad,), jnp.float32),
        grid_spec=pltpu.PrefetchScalarGridSpec(
            num_scalar_prefetch=0,
            grid=(2, g2),
            in_specs=[
                pl.BlockSpec(memory_space=pltpu.VMEM),
                pl.BlockSpec(memory_space=pltpu.VMEM),
                pl.BlockSpec(memory_space=pltpu.VMEM),
            ],
            out_specs=pl.BlockSpec((te,), lambda c, j: (c * g2 + j,)),
            scratch_shapes=[
                pltpu.SMEM((2, 1, 2 * te), jnp.int32),
                pltpu.VMEM((p * te, 128), jnp.float32),
                pltpu.SemaphoreType.DMA((2,)),
            ],
        ),
        compiler_params=pltpu.CompilerParams(
            dimension_semantics=("parallel", "arbitrary")),
        cost_estimate=pl.CostEstimate(
            flops=2 * e_pad * d,
            transcendentals=e_pad,
            bytes_accessed=4 * (2 * n * d + 2 * e_pad * d + 3 * e_pad)),
    )(idx, s4, t4)
    return out[:e]
